# Initial kernel scaffold; baseline (speedup 1.0000x reference)
import jax
import jax.numpy as jnp
from jax.experimental import pallas as pl


def kernel(residues, coordinates, features, mask, edge_index, distances, emb, W1, b1, W2, b2, We1, be1, We2, be2, Wm, bm, Wu1, bu1, Wu2, bu2, Wp1, bp1, Wp2, bp2, gamma, beta):
    raise NotImplementedError("write your pallas kernel here")


# speedup vs baseline: 7.3692x; 7.3692x over previous
import functools

import jax
import jax.numpy as jnp
from jax import lax
from jax.experimental import pallas as pl
from jax.experimental.pallas import tpu as pltpu
from jax.experimental.pallas import tpu_sc as plsc

B, N, E = 4, 2048, 65536
EMBED, HID, OUT, L = 512, 256, 512, 4

RB = 512
EB = 2048
CHK = 128
NSC, NTILE = 2, 16
GPC = B // NSC
EPW = GPC * E // NTILE
RPW = GPC * N // NTILE


def _silu(x):
    return x * jax.nn.sigmoid(x)


def _stage1_body(resf, cf, embp, w1a, w1b, b1, w2, b2, wmt0, h_out, ndp_out):
    oh = (resf[:, :] == lax.broadcasted_iota(jnp.int32, (RB, 32), 1))
    oh = oh.astype(jnp.float32)
    re = jnp.dot(oh, embp[:, :], preferred_element_type=jnp.float32)
    x1 = (jnp.dot(re, w1a[:, :], preferred_element_type=jnp.float32)
          + jnp.dot(cf[:, :], w1b[:, :], preferred_element_type=jnp.float32)
          + b1[:, :])
    h1 = _silu(x1)
    h2 = _silu(jnp.dot(h1, w2[:, :], preferred_element_type=jnp.float32)
               + b2[:, :])
    h_out[:, :] = h2
    ndp_out[:, :] = jnp.dot(h2, wmt0[:, :], preferred_element_type=jnp.float32)


def _stage1(resf, cf, embp, w1a, w1b, b1, w2, b2, wmt0):
    nblk = (B * N) // RB
    full = lambda shape: pl.BlockSpec(shape, lambda i: tuple(0 for _ in shape))
    return pl.pallas_call(
        _stage1_body,
        grid=(nblk,),
        in_specs=[
            pl.BlockSpec((RB, 1), lambda i: (i, 0)),
            pl.BlockSpec((RB, 16), lambda i: (i, 0)),
            full((32, EMBED)),
            full((EMBED, HID)),
            full((16, HID)),
            full((1, HID)),
            full((HID, HID)),
            full((1, HID)),
            full((HID, HID)),
        ],
        out_specs=[
            pl.BlockSpec((RB, HID), lambda i: (i, 0)),
            pl.BlockSpec((RB, HID), lambda i: (i, 0)),
        ],
        out_shape=[
            jax.ShapeDtypeStruct((B * N, HID), jnp.float32),
            jax.ShapeDtypeStruct((B * N, HID), jnp.float32),
        ],
    )(resf, cf, embp, w1a, w1b, b1, w2, b2, wmt0)


def _stage2_body(d, we1, be1, we2, be2, ef_out):
    z = _silu(d[:, :] * we1[:, :] + be1[:, :])
    ef_out[:, :] = _silu(jnp.dot(z, we2[:, :],
                                 preferred_element_type=jnp.float32)
                         + be2[:, :])


def _stage2(dflat, we1, be1, we2, be2):
    nblk = (B * E) // EB
    full = lambda shape: pl.BlockSpec(shape, lambda i: tuple(0 for _ in shape))
    return pl.pallas_call(
        _stage2_body,
        grid=(nblk,),
        in_specs=[
            pl.BlockSpec((EB, 1), lambda i: (i, 0)),
            full((1, 128)),
            full((1, 128)),
            full((128, HID)),
            full((1, HID)),
        ],
        out_specs=pl.BlockSpec((EB, HID), lambda i: (i, 0)),
        out_shape=jax.ShapeDtypeStruct((B * E, HID), jnp.float32),
    )(dflat, we1, be1, we2, be2)


@functools.cache
def _make_sc_gather():
    mesh = plsc.VectorSubcoreMesh(core_axis_name="c", subcore_axis_name="s")

    @functools.partial(
        pl.kernel,
        mesh=mesh,
        out_type=jax.ShapeDtypeStruct((B * E, HID), jnp.float32),
        scratch_types=[
            pltpu.VMEM((CHK,), jnp.int32),
            pltpu.VMEM((CHK, HID), jnp.float32),
            pltpu.SemaphoreType.DMA,
        ],
    )
    def gather_k(ndp_hbm, srcg_hbm, sf_hbm, idx_v, rows_v, sem):
        c = lax.axis_index("c")
        s = lax.axis_index("s")
        base = c * (GPC * E) + s * EPW

        def body(i, _):
            e0 = base + i * CHK
            pltpu.sync_copy(srcg_hbm.at[pl.ds(e0, CHK)], idx_v)
            pltpu.async_copy(ndp_hbm.at[idx_v], rows_v, sem).wait()
            pltpu.sync_copy(rows_v, sf_hbm.at[pl.ds(e0, CHK)])
            return 0

        lax.fori_loop(0, EPW // CHK, body, 0)

    return gather_k


@functools.cache
def _make_sc_scatter():
    mesh = plsc.VectorSubcoreMesh(core_axis_name="c", subcore_axis_name="s")

    @functools.partial(
        pl.kernel,
        mesh=mesh,
        out_type=jax.ShapeDtypeStruct((B * N, HID), jnp.float32),
        scratch_types=[
            pltpu.VMEM((CHK,), jnp.int32),
            pltpu.VMEM((CHK, HID), jnp.float32),
            pltpu.VMEM((RPW, HID), jnp.float32),
        ],
    )
    def scatter_k(msg_hbm, tgtg_hbm, zero_hbm, agg_hbm, idx_v, m_v, z_v):
        c = lax.axis_index("c")
        s = lax.axis_index("s")
        pltpu.sync_copy(zero_hbm, z_v)
        rbase = c * (GPC * N) + s * RPW
        pltpu.sync_copy(z_v, agg_hbm.at[pl.ds(rbase, RPW)])
        plsc.subcore_barrier()
        base = c * (GPC * E) + s * EPW

        def body(i, _):
            e0 = base + i * CHK
            pltpu.sync_copy(tgtg_hbm.at[pl.ds(e0, CHK)], idx_v)
            pltpu.sync_copy(msg_hbm.at[pl.ds(e0, CHK)], m_v)
            pltpu.sync_copy(m_v, agg_hbm.at[idx_v], add=True)
            return 0

        lax.fori_loop(0, EPW // CHK, body, 0)

    return scatter_k


def _stage3_body(sf, ef, w, bb, out):
    out[:, :] = _silu(sf[:, :]
                      + jnp.dot(ef[:, :], w[:, :],
                                preferred_element_type=jnp.float32)
                      + bb[:, :])


def _stage3(sf, ef, w, bb):
    nblk = (B * E) // EB
    full = lambda shape: pl.BlockSpec(shape, lambda i: tuple(0 for _ in shape))
    rspec = pl.BlockSpec((EB, HID), lambda i: (i, 0))
    return pl.pallas_call(
        _stage3_body,
        grid=(nblk,),
        in_specs=[rspec, rspec, full((HID, HID)), full((1, HID))],
        out_specs=rspec,
        out_shape=jax.ShapeDtypeStruct((B * E, HID), jnp.float32),
    )(sf, ef, w, bb)


def _stage4_body(nd, agg, wu1a, wu1b, bu1, wu2, bu2, wmt, nd_out, ndp_out):
    u = _silu(jnp.dot(nd[:, :], wu1a[:, :], preferred_element_type=jnp.float32)
              + jnp.dot(agg[:, :], wu1b[:, :],
                        preferred_element_type=jnp.float32)
              + bu1[:, :])
    nd2 = (nd[:, :]
           + jnp.dot(u, wu2[:, :], preferred_element_type=jnp.float32)
           + bu2[:, :])
    nd_out[:, :] = nd2
    ndp_out[:, :] = jnp.dot(nd2, wmt[:, :], preferred_element_type=jnp.float32)


def _stage4(nd, agg, wu1a, wu1b, bu1, wu2, bu2, wmt):
    nblk = (B * N) // RB
    full = lambda shape: pl.BlockSpec(shape, lambda i: tuple(0 for _ in shape))
    rspec = pl.BlockSpec((RB, HID), lambda i: (i, 0))
    return pl.pallas_call(
        _stage4_body,
        grid=(nblk,),
        in_specs=[rspec, rspec, full((HID, HID)), full((HID, HID)),
                  full((1, HID)), full((HID, HID)), full((1, HID)),
                  full((HID, HID))],
        out_specs=[rspec, rspec],
        out_shape=[
            jax.ShapeDtypeStruct((B * N, HID), jnp.float32),
            jax.ShapeDtypeStruct((B * N, HID), jnp.float32),
        ],
    )(nd, agg, wu1a, wu1b, bu1, wu2, bu2, wmt)


def _stage5_body(nd, m, wp1, bp1, wp2, bp2, gamma, beta, out):
    ndm = nd[:, :] * m[0, :, :]
    pooled = jnp.sum(ndm, axis=0, keepdims=True)
    msum = jnp.sum(m[0, :, :])
    pooled = pooled / jnp.maximum(msum, jnp.float32(1.0))
    h = _silu(jnp.dot(pooled, wp1[:, :], preferred_element_type=jnp.float32)
              + bp1[:, :])
    o = jnp.dot(h, wp2[:, :], preferred_element_type=jnp.float32) + bp2[:, :]
    mu = jnp.mean(o, axis=-1, keepdims=True)
    var = jnp.mean((o - mu) ** 2, axis=-1, keepdims=True)
    o = (o - mu) * lax.rsqrt(var + 1e-5) * gamma[:, :] + beta[:, :]
    out[0, :, :] = o


def _stage5(nd, m3, wp1, bp1, wp2, bp2, gamma, beta):
    full = lambda shape: pl.BlockSpec(shape, lambda i: tuple(0 for _ in shape))
    return pl.pallas_call(
        _stage5_body,
        grid=(B,),
        in_specs=[
            pl.BlockSpec((N, HID), lambda i: (i, 0)),
            pl.BlockSpec((1, N, 1), lambda i: (i, 0, 0)),
            full((HID, HID)),
            full((1, HID)),
            full((HID, OUT)),
            full((1, OUT)),
            full((1, OUT)),
            full((1, OUT)),
        ],
        out_specs=pl.BlockSpec((1, 1, OUT), lambda i: (i, 0, 0)),
        out_shape=jax.ShapeDtypeStruct((B, 1, OUT), jnp.float32),
    )(nd, m3, wp1, bp1, wp2, bp2, gamma, beta)


def kernel(residues, coordinates, features, mask, edge_index, distances,
           emb, W1, b1, W2, b2, We1, be1, We2, be2,
           Wm, bm, Wu1, bu1, Wu2, bu2, Wp1, bp1, Wp2, bp2, gamma, beta):
    f32 = jnp.float32
    resf = residues.astype(jnp.int32).reshape(B * N, 1)
    cf = jnp.concatenate([coordinates, features], axis=-1).astype(f32)
    cf = jnp.pad(cf, ((0, 0), (0, 0), (0, 4))).reshape(B * N, 16)
    embp = jnp.pad(emb.astype(f32), ((0, 32 - emb.shape[0]), (0, 0)))
    w1a = W1[:EMBED].astype(f32)
    w1b = jnp.pad(W1[EMBED:].astype(f32), ((0, 4), (0, 0)))
    b1r = b1.astype(f32).reshape(1, HID)
    b2r = b2.astype(f32).reshape(1, HID)
    wmt = [Wm[l, :HID].astype(f32) for l in range(L)]
    wmb = [Wm[l, HID:].astype(f32) for l in range(L)]
    bmr = [bm[l].astype(f32).reshape(1, HID) for l in range(L)]

    h, ndp = _stage1(resf, cf, embp, w1a, w1b, b1r, W2.astype(f32),
                     b2r, wmt[0])

    dflat = distances.astype(f32).reshape(B * E, 1)
    ef = _stage2(dflat, We1.astype(f32), be1.astype(f32).reshape(1, 128),
                 We2.astype(f32), be2.astype(f32).reshape(1, HID))

    goff = (jnp.arange(B, dtype=jnp.int32) * N)[:, None]
    srcg = (edge_index[:, 0, :].astype(jnp.int32) + goff).reshape(B * E)
    tgtg = (edge_index[:, 1, :].astype(jnp.int32) + goff).reshape(B * E)
    zero = jnp.zeros((RPW, HID), f32)

    nd = h
    for l in range(L):
        sf = _make_sc_gather()(ndp, srcg)
        msg = _stage3(sf, ef, wmb[l], bmr[l])
        agg = _make_sc_scatter()(msg, tgtg, zero)
        wmt_next = wmt[l + 1] if l + 1 < L else jnp.zeros((HID, HID), f32)
        nd, ndp = _stage4(nd, agg, Wu1[l, :HID].astype(f32),
                          Wu1[l, HID:].astype(f32),
                          bu1[l].astype(f32).reshape(1, HID),
                          Wu2[l].astype(f32),
                          bu2[l].astype(f32).reshape(1, HID), wmt_next)

    out = _stage5(nd, mask.astype(f32).reshape(B, N, 1),
                  Wp1.astype(f32), bp1.astype(f32).reshape(1, HID),
                  Wp2.astype(f32), bp2.astype(f32).reshape(1, OUT),
                  gamma.astype(f32).reshape(1, OUT),
                  beta.astype(f32).reshape(1, OUT))
    return out.reshape(B, OUT)
